# baseline (speedup 1.0000x reference)
import functools

import jax
import jax.numpy as jnp
import numpy as np
from jax import lax
from jax.experimental import pallas as pl
from jax.experimental.pallas import tpu as pltpu
from jax.experimental.pallas import tpu_sc as plsc

NN = 10000
NE = 320000
D = 128
C = 128
NCH = NE // C
NW = 32
JMAX = -(-NCH // NW)
JSTORE = 96
JG = 81
NSUB = 16
DP = D // 2

_ORDER = np.minimum(
    np.arange(NW)[:, None] + NW * np.arange(JSTORE)[None, :], NCH - 1
).reshape(-1)
ZR = 624
ZTAIL = NN - NSUB * ZR

_mesh = plsc.VectorSubcoreMesh(core_axis_name="c", subcore_axis_name="s")


@functools.partial(
    pl.kernel,
    out_type=jax.ShapeDtypeStruct((2 * NN, D), jnp.float32),
    mesh=_mesh,
    scratch_types=[
        pltpu.VMEM((JSTORE, C), jnp.int32),
        pltpu.VMEM((C, D), jnp.float32),
        pltpu.VMEM((C, D), jnp.float32),
        pltpu.VMEM_SHARED((NN, D), jnp.float32),
        pltpu.SemaphoreType.DMA,
        pltpu.SemaphoreType.DMA,
        pltpu.SemaphoreType.DMA,
        pltpu.SemaphoreType.DMA,
    ],
)
def _sc_scatter(edges_hbm, dstord_hbm, zeros_hbm, out_hbm,
                idx_all, ed0, ed1, acc_sh, sem0, sem1, sas0, sas1):
    c = lax.axis_index("c")
    s = lax.axis_index("s")
    w = s * 2 + c
    ed = [ed0, ed1]
    sem = [sem0, sem1]
    sas = [sas0, sas1]
    pltpu.sync_copy(zeros_hbm.at[pl.ds(s * ZR, ZR)],
                    acc_sh.at[pl.ds(s * ZR, ZR)])

    @pl.when(s == NSUB - 1)
    def _():
        pltpu.sync_copy(zeros_hbm.at[pl.ds(NSUB * ZR, ZTAIL)],
                        acc_sh.at[pl.ds(NSUB * ZR, ZTAIL)])

    pltpu.sync_copy(dstord_hbm.at[pl.ds(w * JSTORE, JSTORE)], idx_all)
    plsc.subcore_barrier()

    def issue(j, p):
        k = w + NW * j
        pltpu.async_copy(edges_hbm.at[pl.ds(k * C, C)], ed[p], sem[p])

    def wait(j, p):
        k = w + NW * j
        pltpu.make_async_copy(edges_hbm.at[pl.ds(k * C, C)], ed[p],
                              sem[p]).wait()

    def sadd(j, p):
        pltpu.async_copy(ed[p], acc_sh.at[idx_all.at[j]], sas[p], add=True)

    def wait_sadd(j, p):
        pltpu.make_async_copy(ed[p], acc_sh.at[idx_all.at[j]],
                              sas[p]).wait()

    issue(0, 0)

    def outer(t, carry):
        for b in range(2):
            j = 2 * t + b
            p = b
            wait(j, p)

            @pl.when(j >= 1)
            def _():
                wait_sadd(j - 1, 1 - p)

            @pl.when(w + NW * (j + 1) < NCH)
            def _():
                issue(j + 1, 1 - p)

            sadd(j, p)
        return carry

    lax.fori_loop(0, (JMAX - 1) // 2, outer, 0)
    wait_sadd(JMAX - 2, 1)

    @pl.when(w + NW * (JMAX - 1) < NCH)
    def _():
        wait(JMAX - 1, 0)
        pltpu.sync_copy(ed[0], acc_sh.at[idx_all.at[JMAX - 1]], add=True)

    plsc.subcore_barrier()
    pltpu.sync_copy(acc_sh.at[pl.ds(s * ZR, ZR)],
                    out_hbm.at[pl.ds(c * NN + s * ZR, ZR)])

    @pl.when(s == NSUB - 1)
    def _():
        pltpu.sync_copy(acc_sh.at[pl.ds(NSUB * ZR, ZTAIL)],
                        out_hbm.at[pl.ds(c * NN + NSUB * ZR, ZTAIL)])


@functools.partial(
    pl.kernel,
    out_type=jax.ShapeDtypeStruct((NE, D), jnp.float32),
    mesh=_mesh,
    scratch_types=[
        pltpu.VMEM((JSTORE, C), jnp.int32),
        pltpu.VMEM((JSTORE, C), jnp.int32),
        pltpu.VMEM((C, D), jnp.float32),
        pltpu.VMEM((C, D), jnp.float32),
        pltpu.VMEM((C, D), jnp.float32),
        pltpu.VMEM((C, D), jnp.float32),
        pltpu.VMEM((C, D), jnp.float32),
        pltpu.VMEM((C, D), jnp.float32),
        pltpu.SemaphoreType.DMA,
        pltpu.SemaphoreType.DMA,
        pltpu.SemaphoreType.DMA,
        pltpu.SemaphoreType.DMA,
        pltpu.SemaphoreType.DMA,
        pltpu.SemaphoreType.DMA,
        pltpu.SemaphoreType.DMA,
        pltpu.SemaphoreType.DMA,
        pltpu.SemaphoreType.DMA,
    ],
)
def _sc_gather(a_hbm, b_hbm, srcord_hbm, dstord_hbm, out_hbm,
               idxa_all, idxb_all, bufa0, bufa1, bufa2, bufb0, bufb1, bufb2,
               sema0, sema1, sema2, semb0, semb1, semb2, semo0, semo1, semo2):
    c = lax.axis_index("c")
    s = lax.axis_index("s")
    w = s * 2 + c
    bufa = [bufa0, bufa1, bufa2]
    bufb = [bufb0, bufb1, bufb2]
    sema = [sema0, sema1, sema2]
    semb = [semb0, semb1, semb2]
    semo = [semo0, semo1, semo2]

    pltpu.sync_copy(srcord_hbm.at[pl.ds(w * JSTORE, JSTORE)], idxa_all)
    pltpu.sync_copy(dstord_hbm.at[pl.ds(w * JSTORE, JSTORE)], idxb_all)

    def kof(j):
        return jnp.minimum(w + NW * j, NCH - 1)

    def issue(j, p):
        pltpu.async_copy(a_hbm.at[idxa_all.at[j]], bufa[p], sema[p])
        pltpu.async_copy(b_hbm.at[idxb_all.at[j]], bufb[p], semb[p])

    def wait(j, p):
        pltpu.make_async_copy(a_hbm.at[idxa_all.at[j]], bufa[p],
                              sema[p]).wait()
        pltpu.make_async_copy(b_hbm.at[idxb_all.at[j]], bufb[p],
                              semb[p]).wait()

    def wait_out(j, p):
        pltpu.make_async_copy(
            bufa[p], out_hbm.at[pl.ds(kof(j) * C, C)], semo[p]).wait()

    issue(0, 0)
    issue(1, 1)

    def outer(t, carry):
        for b in range(3):
            j = 3 * t + b
            p = b
            pnext = (p + 2) % 3
            wait(j, p)

            @pl.when(j + 2 < JG)
            def _():
                @pl.when(j >= 1)
                def _():
                    wait_out(j - 1, pnext)

                issue(j + 2, pnext)

            def addrow(r, cr):
                for u in range(D // 16):
                    plsc.addupdate(bufa[p].at[r, pl.ds(u * 16, 16)],
                                   bufb[p][r, pl.ds(u * 16, 16)])
                return cr

            lax.fori_loop(0, C, addrow, 0)
            pltpu.async_copy(bufa[p], out_hbm.at[pl.ds(kof(j) * C, C)],
                             semo[p])
        return carry

    lax.fori_loop(0, JG // 3, outer, 0)
    wait_out(JG - 3, 0)
    wait_out(JG - 2, 1)
    wait_out(JG - 1, 2)


def _ln(x, g, b):
    m = jnp.mean(x, axis=-1, keepdims=True)
    xc = x - m
    v = jnp.mean(xc * xc, axis=-1, keepdims=True)
    return xc * lax.rsqrt(v + 1e-5) * g + b


def _silu(x):
    return x * jax.nn.sigmoid(x)


def _unpack_bf16_pairs(xp, nrows):
    u = jax.lax.bitcast_convert_type(xp, jnp.uint32)
    lo = jax.lax.bitcast_convert_type(u << 16, jnp.float32)
    hi = jax.lax.bitcast_convert_type(u & jnp.uint32(0xFFFF0000), jnp.float32)
    return jnp.concatenate([lo.reshape(nrows, DP), hi.reshape(nrows, DP)],
                           axis=1)


def _node_body(p_ref, n_ref, w1a, w1b, b1, g1, bb1, w2, b2, g2, bb2, ws, wd,
               nn_ref, a_ref, b_ref):
    msg = p_ref[0:NN, :] + p_ref[NN:2 * NN, :]
    nodes = n_ref[...]
    x = (jnp.dot(nodes, w1a[...], preferred_element_type=jnp.float32)
         + jnp.dot(msg, w1b[...], preferred_element_type=jnp.float32)
         + b1[...])
    x = _silu(_ln(x, g1[...], bb1[...]))
    x = jnp.dot(x, w2[...], preferred_element_type=jnp.float32) + b2[...]
    x = _silu(_ln(x, g2[...], bb2[...]))
    nn = x + nodes
    nn_ref[...] = nn
    a_ref[...] = jnp.dot(nn, ws[...], preferred_element_type=jnp.float32)
    b_ref[...] = jnp.dot(nn, wd[...], preferred_element_type=jnp.float32)


BLK = 2000


def _edge_body(g_ref, e_ref, we, b1, g1, bb1, w2, b2, g2, bb2, out_ref):
    e = e_ref[...]
    h = (g_ref[...]
         + jnp.dot(e, we[...], preferred_element_type=jnp.float32)
         + b1[...])
    h = _silu(_ln(h, g1[...], bb1[...]))
    h = jnp.dot(h, w2[...], preferred_element_type=jnp.float32) + b2[...]
    h = _ln(h, g2[...], bb2[...])
    out_ref[...] = jnp.tanh(h) + e


def _row2d(v):
    return v.reshape(1, D)


def kernel(nodes, edges, node_params, edge_params, graph):
    graph = graph.astype(jnp.int32)
    order = jnp.asarray(_ORDER, dtype=jnp.int32)
    srcord = jnp.take(graph[0].reshape(NCH, C), order, axis=0)
    dstord = jnp.take(graph[1].reshape(NCH, C), order, axis=0)
    zeros = jnp.zeros((NN, D), jnp.float32)

    partials = _sc_scatter(edges, dstord, zeros)

    np0, np1 = node_params
    ep0, ep1 = edge_params
    w1a, w1b = np0['W'][:D], np0['W'][D:]
    ws, wd, we = ep0['W'][:D], ep0['W'][D:2 * D], ep0['W'][2 * D:]

    full = pl.BlockSpec((D, D), lambda i: (0, 0))
    row = pl.BlockSpec((1, D), lambda i: (0, 0))

    nodes_new, a_arr, b_arr = pl.pallas_call(
        _node_body,
        out_shape=[jax.ShapeDtypeStruct((NN, D), jnp.float32)] * 3,
    )(partials, nodes, w1a, w1b, _row2d(np0['b']), _row2d(np0['g']),
      _row2d(np0['beta']), np1['W'], _row2d(np1['b']), _row2d(np1['g']),
      _row2d(np1['beta']), ws, wd)

    g_arr = _sc_gather(a_arr, b_arr, srcord, dstord)

    blk = pl.BlockSpec((BLK, D), lambda i: (i, 0))
    blkp = blk
    edges_new = pl.pallas_call(
        _edge_body,
        grid=(NE // BLK,),
        in_specs=[blkp, blk, full, row, row, row, full, row, row, row],
        out_specs=blk,
        out_shape=jax.ShapeDtypeStruct((NE, D), jnp.float32),
    )(g_arr, edges, we, _row2d(ep0['b']), _row2d(ep0['g']),
      _row2d(ep0['beta']), ep1['W'], _row2d(ep1['b']), _row2d(ep1['g']),
      _row2d(ep1['beta']))

    return nodes_new, edges_new


# speedup vs baseline: 1.0937x; 1.0937x over previous
import functools

import jax
import jax.numpy as jnp
import numpy as np
from jax import lax
from jax.experimental import pallas as pl
from jax.experimental.pallas import tpu as pltpu
from jax.experimental.pallas import tpu_sc as plsc

NN = 10000
NE = 320000
D = 128
C = 128
NCH = NE // C
NW = 32
JMAX = -(-NCH // NW)
JSTORE = 96
NSUB = 16
NCHH = NCH // 2
JH = 40

_ORDER = np.minimum(
    np.arange(NW)[:, None] + NW * np.arange(JSTORE)[None, :], NCH - 1
).reshape(-1)
_ORDER_G = np.concatenate([
    (np.minimum(np.arange(NW)[:, None] + NW * np.arange(JH)[None, :],
                NCHH - 1) + NCHH * h).reshape(-1)
    for h in (0, 1)
])
ZR = 624
ZTAIL = NN - NSUB * ZR

_mesh = plsc.VectorSubcoreMesh(core_axis_name="c", subcore_axis_name="s")


@functools.partial(
    pl.kernel,
    out_type=jax.ShapeDtypeStruct((2 * NN, D), jnp.float32),
    mesh=_mesh,
    scratch_types=[
        pltpu.VMEM((JSTORE, C), jnp.int32),
        pltpu.VMEM((C, D), jnp.float32),
        pltpu.VMEM((C, D), jnp.float32),
        pltpu.VMEM_SHARED((NN, D), jnp.float32),
        pltpu.SemaphoreType.DMA,
        pltpu.SemaphoreType.DMA,
    ],
)
def _sc_scatter(edges_hbm, dstord_hbm, zeros_hbm, out_hbm,
                idx_all, ed0, ed1, acc_sh, sem0, sem1):
    c = lax.axis_index("c")
    s = lax.axis_index("s")
    w = s * 2 + c
    ed = [ed0, ed1]
    sem = [sem0, sem1]
    pltpu.sync_copy(zeros_hbm.at[pl.ds(s * ZR, ZR)],
                    acc_sh.at[pl.ds(s * ZR, ZR)])

    @pl.when(s == NSUB - 1)
    def _():
        pltpu.sync_copy(zeros_hbm.at[pl.ds(NSUB * ZR, ZTAIL)],
                        acc_sh.at[pl.ds(NSUB * ZR, ZTAIL)])

    pltpu.sync_copy(dstord_hbm.at[pl.ds(w * JSTORE, JSTORE)], idx_all)
    plsc.subcore_barrier()

    def issue(j, p):
        k = w + NW * j
        pltpu.async_copy(edges_hbm.at[pl.ds(k * C, C)], ed[p], sem[p])

    def wait(j, p):
        k = w + NW * j
        pltpu.make_async_copy(edges_hbm.at[pl.ds(k * C, C)], ed[p],
                              sem[p]).wait()

    issue(0, 0)

    def outer(t, carry):
        for b in range(2):
            j = 2 * t + b
            p = b
            wait(j, p)

            @pl.when(w + NW * (j + 1) < NCH)
            def _():
                issue(j + 1, 1 - p)

            pltpu.sync_copy(ed[p], acc_sh.at[idx_all.at[j]], add=True)
        return carry

    lax.fori_loop(0, (JMAX - 1) // 2, outer, 0)

    @pl.when(w + NW * (JMAX - 1) < NCH)
    def _():
        wait(JMAX - 1, 0)
        pltpu.sync_copy(ed[0], acc_sh.at[idx_all.at[JMAX - 1]], add=True)

    plsc.subcore_barrier()
    pltpu.sync_copy(acc_sh.at[pl.ds(s * ZR, ZR)],
                    out_hbm.at[pl.ds(c * NN + s * ZR, ZR)])

    @pl.when(s == NSUB - 1)
    def _():
        pltpu.sync_copy(acc_sh.at[pl.ds(NSUB * ZR, ZTAIL)],
                        out_hbm.at[pl.ds(c * NN + NSUB * ZR, ZTAIL)])


def _make_gather(h):
    off = h * NW * JH

    @functools.partial(
        pl.kernel,
        out_type=jax.ShapeDtypeStruct((NE // 2, D), jnp.float32),
        mesh=_mesh,
        scratch_types=[
            pltpu.VMEM((JH, C), jnp.int32),
            pltpu.VMEM((JH, C), jnp.int32),
            pltpu.VMEM((C, D), jnp.float32),
            pltpu.VMEM((C, D), jnp.float32),
            pltpu.VMEM((C, D), jnp.float32),
            pltpu.VMEM((C, D), jnp.float32),
            pltpu.SemaphoreType.DMA,
            pltpu.SemaphoreType.DMA,
            pltpu.SemaphoreType.DMA,
            pltpu.SemaphoreType.DMA,
            pltpu.SemaphoreType.DMA,
            pltpu.SemaphoreType.DMA,
        ],
    )
    def gather_k(a_hbm, b_hbm, srcord_hbm, dstord_hbm, out_hbm,
                 idxa_all, idxb_all, bufa0, bufa1, bufb0, bufb1,
                 sema0, sema1, semb0, semb1, semo0, semo1):
        c = lax.axis_index("c")
        s = lax.axis_index("s")
        w = s * 2 + c
        bufa = [bufa0, bufa1]
        bufb = [bufb0, bufb1]
        sema = [sema0, sema1]
        semb = [semb0, semb1]
        semo = [semo0, semo1]

        pltpu.sync_copy(srcord_hbm.at[pl.ds(off + w * JH, JH)], idxa_all)
        pltpu.sync_copy(dstord_hbm.at[pl.ds(off + w * JH, JH)], idxb_all)

        def kof(j):
            return jnp.minimum(w + NW * j, NCHH - 1)

        def issue(j, p):
            pltpu.async_copy(a_hbm.at[idxa_all.at[j]], bufa[p], sema[p])
            pltpu.async_copy(b_hbm.at[idxb_all.at[j]], bufb[p], semb[p])

        def wait(j, p):
            pltpu.make_async_copy(a_hbm.at[idxa_all.at[j]], bufa[p],
                                  sema[p]).wait()
            pltpu.make_async_copy(b_hbm.at[idxb_all.at[j]], bufb[p],
                                  semb[p]).wait()

        def wait_out(j, p):
            pltpu.make_async_copy(
                bufa[p], out_hbm.at[pl.ds(kof(j) * C, C)], semo[p]).wait()

        issue(0, 0)

        def outer(t, carry):
            for b in range(2):
                j = 2 * t + b
                p = b
                wait(j, p)

                @pl.when(j + 1 < JH)
                def _():
                    @pl.when(j >= 1)
                    def _():
                        wait_out(j - 1, 1 - p)

                    issue(j + 1, 1 - p)

                def addrow(r, cr):
                    for u in range(D // 16):
                        plsc.addupdate(bufa[p].at[r, pl.ds(u * 16, 16)],
                                       bufb[p][r, pl.ds(u * 16, 16)])
                    return cr

                lax.fori_loop(0, C, addrow, 0)
                pltpu.async_copy(bufa[p], out_hbm.at[pl.ds(kof(j) * C, C)],
                                 semo[p])
            return carry

        lax.fori_loop(0, JH // 2, outer, 0)
        wait_out(JH - 2, 0)
        wait_out(JH - 1, 1)

    return gather_k


_sc_gather0 = _make_gather(0)
_sc_gather1 = _make_gather(1)


def _ln(x, g, b):
    m = jnp.mean(x, axis=-1, keepdims=True)
    xc = x - m
    v = jnp.mean(xc * xc, axis=-1, keepdims=True)
    return xc * lax.rsqrt(v + 1e-5) * g + b


def _silu(x):
    return x * jax.nn.sigmoid(x)


def _unpack_bf16_pairs(xp, nrows):
    u = jax.lax.bitcast_convert_type(xp, jnp.uint32)
    lo = jax.lax.bitcast_convert_type(u << 16, jnp.float32)
    hi = jax.lax.bitcast_convert_type(u & jnp.uint32(0xFFFF0000), jnp.float32)
    return jnp.concatenate([lo.reshape(nrows, DP), hi.reshape(nrows, DP)],
                           axis=1)


def _node_body(p_ref, n_ref, w1a, w1b, b1, g1, bb1, w2, b2, g2, bb2, ws, wd,
               nn_ref, a_ref, b_ref):
    msg = p_ref[0:NN, :] + p_ref[NN:2 * NN, :]
    nodes = n_ref[...]
    x = (jnp.dot(nodes, w1a[...], preferred_element_type=jnp.float32)
         + jnp.dot(msg, w1b[...], preferred_element_type=jnp.float32)
         + b1[...])
    x = _silu(_ln(x, g1[...], bb1[...]))
    x = jnp.dot(x, w2[...], preferred_element_type=jnp.float32) + b2[...]
    x = _silu(_ln(x, g2[...], bb2[...]))
    nn = x + nodes
    nn_ref[...] = nn
    a_ref[...] = jnp.dot(nn, ws[...], preferred_element_type=jnp.float32)
    b_ref[...] = jnp.dot(nn, wd[...], preferred_element_type=jnp.float32)


BLK = 2000


def _edge_body(g_ref, e_ref, we, b1, g1, bb1, w2, b2, g2, bb2, out_ref):
    e = e_ref[...]
    h = (g_ref[...]
         + jnp.dot(e, we[...], preferred_element_type=jnp.float32)
         + b1[...])
    h = _silu(_ln(h, g1[...], bb1[...]))
    h = jnp.dot(h, w2[...], preferred_element_type=jnp.float32) + b2[...]
    h = _ln(h, g2[...], bb2[...])
    out_ref[...] = jnp.tanh(h) + e


def _edge_body2(car_ref, g_ref, e_ref, we, b1, g1, bb1, w2, b2, g2, bb2,
                out_ref):
    del car_ref
    _edge_body(g_ref, e_ref, we, b1, g1, bb1, w2, b2, g2, bb2, out_ref)


def _row2d(v):
    return v.reshape(1, D)


def kernel(nodes, edges, node_params, edge_params, graph):
    graph = graph.astype(jnp.int32)
    src2d = graph[0].reshape(NCH, C)
    dst2d = graph[1].reshape(NCH, C)
    order_s = jnp.asarray(_ORDER, dtype=jnp.int32)
    order_g = jnp.asarray(_ORDER_G, dtype=jnp.int32)
    dstord_s = jnp.take(dst2d, order_s, axis=0)
    srcord_g = jnp.take(src2d, order_g, axis=0)
    dstord_g = jnp.take(dst2d, order_g, axis=0)
    zeros = jnp.zeros((NN, D), jnp.float32)

    partials = _sc_scatter(edges, dstord_s, zeros)

    np0, np1 = node_params
    ep0, ep1 = edge_params
    w1a, w1b = np0['W'][:D], np0['W'][D:]
    ws, wd, we = ep0['W'][:D], ep0['W'][D:2 * D], ep0['W'][2 * D:]

    full = pl.BlockSpec((D, D), lambda i: (0, 0))
    row = pl.BlockSpec((1, D), lambda i: (0, 0))

    nodes_new, a_arr, b_arr = pl.pallas_call(
        _node_body,
        out_shape=[jax.ShapeDtypeStruct((NN, D), jnp.float32)] * 3,
    )(partials, nodes, w1a, w1b, _row2d(np0['b']), _row2d(np0['g']),
      _row2d(np0['beta']), np1['W'], _row2d(np1['b']), _row2d(np1['g']),
      _row2d(np1['beta']), ws, wd)

    g0 = _sc_gather0(a_arr, b_arr, srcord_g, dstord_g)
    g1 = _sc_gather1(a_arr, b_arr, srcord_g, dstord_g)

    nblk_h = (NE // 2) // BLK
    blk = pl.BlockSpec((BLK, D), lambda i: (i, 0))
    blk_hi = pl.BlockSpec((BLK, D), lambda i: (i + nblk_h, 0))
    ewts = (we, _row2d(ep0['b']), _row2d(ep0['g']), _row2d(ep0['beta']),
            ep1['W'], _row2d(ep1['b']), _row2d(ep1['g']), _row2d(ep1['beta']))
    wspecs = [full, row, row, row, full, row, row, row]

    o0 = pl.pallas_call(
        _edge_body,
        grid=(nblk_h,),
        in_specs=[blk, blk] + wspecs,
        out_specs=blk,
        out_shape=jax.ShapeDtypeStruct((NE, D), jnp.float32),
    )(g0, edges, *ewts)

    edges_new = pl.pallas_call(
        _edge_body2,
        grid=(nblk_h,),
        in_specs=[pl.BlockSpec(memory_space=pl.ANY), blk, blk_hi] + wspecs,
        out_specs=blk_hi,
        out_shape=jax.ShapeDtypeStruct((NE, D), jnp.float32),
        input_output_aliases={0: 0},
    )(o0, g1, edges, *ewts)

    return nodes_new, edges_new


# speedup vs baseline: 1.1737x; 1.0732x over previous
import functools

import jax
import jax.numpy as jnp
import numpy as np
from jax import lax
from jax.experimental import pallas as pl
from jax.experimental.pallas import tpu as pltpu
from jax.experimental.pallas import tpu_sc as plsc

NN = 10000
NE = 320000
D = 128
C = 128
NCH = NE // C
NW = 32
JMAX = -(-NCH // NW)
NSUB = 16
NCHH = NCH // 2
JH = 40
ZR = 624
ZTAIL = NN - NSUB * ZR

_mesh = plsc.VectorSubcoreMesh(core_axis_name="c", subcore_axis_name="s")


@functools.partial(
    pl.kernel,
    out_type=jax.ShapeDtypeStruct((2 * NN, D), jnp.float32),
    mesh=_mesh,
    scratch_types=[
        pltpu.VMEM((1, C), jnp.int32),
        pltpu.VMEM((1, C), jnp.int32),
        pltpu.VMEM((C, D), jnp.float32),
        pltpu.VMEM((C, D), jnp.float32),
        pltpu.VMEM_SHARED((NN, D), jnp.float32),
        pltpu.SemaphoreType.DMA,
        pltpu.SemaphoreType.DMA,
        pltpu.SemaphoreType.DMA,
        pltpu.SemaphoreType.DMA,
    ],
)
def _sc_scatter(edges_hbm, dst3d_hbm, zeros_hbm, out_hbm,
                idx0, idx1, ed0, ed1, acc_sh, sem0, sem1, isem0, isem1):
    c = lax.axis_index("c")
    s = lax.axis_index("s")
    w = s * 2 + c
    ed = [ed0, ed1]
    sem = [sem0, sem1]
    idx = [idx0, idx1]
    isem = [isem0, isem1]
    pltpu.sync_copy(zeros_hbm.at[pl.ds(s * ZR, ZR)],
                    acc_sh.at[pl.ds(s * ZR, ZR)])

    @pl.when(s == NSUB - 1)
    def _():
        pltpu.sync_copy(zeros_hbm.at[pl.ds(NSUB * ZR, ZTAIL)],
                        acc_sh.at[pl.ds(NSUB * ZR, ZTAIL)])

    plsc.subcore_barrier()

    def issue(j, p):
        k = w + NW * j
        pltpu.async_copy(edges_hbm.at[pl.ds(k * C, C)], ed[p], sem[p])

    def wait(j, p):
        k = w + NW * j
        pltpu.make_async_copy(edges_hbm.at[pl.ds(k * C, C)], ed[p],
                              sem[p]).wait()

    def issue_idx(j, p):
        pltpu.async_copy(dst3d_hbm.at[w + NW * j], idx[p], isem[p])

    def wait_idx(j, p):
        pltpu.make_async_copy(dst3d_hbm.at[w + NW * j], idx[p],
                              isem[p]).wait()

    pltpu.sync_copy(dst3d_hbm.at[w], idx[0])
    issue(0, 0)
    issue_idx(1, 1)

    def outer(t, carry):
        for b in range(2):
            j = 2 * t + b
            p = b
            wait(j, p)

            @pl.when(w + NW * (j + 1) < NCH)
            def _():
                wait_idx(j + 1, 1 - p)
                issue(j + 1, 1 - p)

            pltpu.sync_copy(ed[p], acc_sh.at[idx[p].at[0]], add=True)

            @pl.when(w + NW * (j + 2) < NCH)
            def _():
                issue_idx(j + 2, p)
        return carry

    lax.fori_loop(0, (JMAX - 1) // 2, outer, 0)

    @pl.when(w + NW * (JMAX - 1) < NCH)
    def _():
        wait(JMAX - 1, 0)
        pltpu.sync_copy(ed[0], acc_sh.at[idx[0].at[0]], add=True)

    plsc.subcore_barrier()
    pltpu.sync_copy(acc_sh.at[pl.ds(s * ZR, ZR)],
                    out_hbm.at[pl.ds(c * NN + s * ZR, ZR)])

    @pl.when(s == NSUB - 1)
    def _():
        pltpu.sync_copy(acc_sh.at[pl.ds(NSUB * ZR, ZTAIL)],
                        out_hbm.at[pl.ds(c * NN + NSUB * ZR, ZTAIL)])


def _make_gather(h):
    kb = h * NCHH

    @functools.partial(
        pl.kernel,
        out_type=jax.ShapeDtypeStruct((NE // 2, D), jnp.float32),
        mesh=_mesh,
        scratch_types=[
            pltpu.VMEM((1, C), jnp.int32),
            pltpu.VMEM((1, C), jnp.int32),
            pltpu.VMEM((1, C), jnp.int32),
            pltpu.VMEM((1, C), jnp.int32),
            pltpu.VMEM((C, D), jnp.float32),
            pltpu.VMEM((C, D), jnp.float32),
            pltpu.VMEM((C, D), jnp.float32),
            pltpu.VMEM((C, D), jnp.float32),
            pltpu.SemaphoreType.DMA,
            pltpu.SemaphoreType.DMA,
            pltpu.SemaphoreType.DMA,
            pltpu.SemaphoreType.DMA,
            pltpu.SemaphoreType.DMA,
            pltpu.SemaphoreType.DMA,
            pltpu.SemaphoreType.DMA,
            pltpu.SemaphoreType.DMA,
            pltpu.SemaphoreType.DMA,
            pltpu.SemaphoreType.DMA,
        ],
    )
    def gather_k(a_hbm, b_hbm, src3d_hbm, dst3d_hbm, out_hbm,
                 idxa0, idxa1, idxb0, idxb1, bufa0, bufa1, bufb0, bufb1,
                 sema0, sema1, semb0, semb1, semo0, semo1,
                 isa0, isa1, isb0, isb1):
        c = lax.axis_index("c")
        s = lax.axis_index("s")
        w = s * 2 + c
        idxa = [idxa0, idxa1]
        idxb = [idxb0, idxb1]
        bufa = [bufa0, bufa1]
        bufb = [bufb0, bufb1]
        sema = [sema0, sema1]
        semb = [semb0, semb1]
        semo = [semo0, semo1]
        isa = [isa0, isa1]
        isb = [isb0, isb1]

        def kof(j):
            return jnp.minimum(w + NW * j, NCHH - 1)

        def issue_idx(j, p):
            pltpu.async_copy(src3d_hbm.at[kb + kof(j)], idxa[p], isa[p])
            pltpu.async_copy(dst3d_hbm.at[kb + kof(j)], idxb[p], isb[p])

        def wait_idx(j, p):
            pltpu.make_async_copy(src3d_hbm.at[kb + kof(j)], idxa[p],
                                  isa[p]).wait()
            pltpu.make_async_copy(dst3d_hbm.at[kb + kof(j)], idxb[p],
                                  isb[p]).wait()

        def issue(j, p):
            pltpu.async_copy(a_hbm.at[idxa[p].at[0]], bufa[p], sema[p])
            pltpu.async_copy(b_hbm.at[idxb[p].at[0]], bufb[p], semb[p])

        def wait(j, p):
            pltpu.make_async_copy(a_hbm.at[idxa[p].at[0]], bufa[p],
                                  sema[p]).wait()
            pltpu.make_async_copy(b_hbm.at[idxb[p].at[0]], bufb[p],
                                  semb[p]).wait()

        def wait_out(j, p):
            pltpu.make_async_copy(
                bufa[p], out_hbm.at[pl.ds(kof(j) * C, C)], semo[p]).wait()

        pltpu.sync_copy(src3d_hbm.at[kb + kof(0)], idxa[0])
        pltpu.sync_copy(dst3d_hbm.at[kb + kof(0)], idxb[0])
        issue(0, 0)
        issue_idx(1, 1)

        def outer(t, carry):
            for b in range(2):
                j = 2 * t + b
                p = b
                wait(j, p)

                @pl.when(j + 1 < JH)
                def _():
                    @pl.when(j >= 1)
                    def _():
                        wait_out(j - 1, 1 - p)

                    wait_idx(j + 1, 1 - p)
                    issue(j + 1, 1 - p)

                @pl.when(j + 2 < JH)
                def _():
                    issue_idx(j + 2, p)

                def addrow(r, cr):
                    for u in range(D // 16):
                        plsc.addupdate(bufa[p].at[r, pl.ds(u * 16, 16)],
                                       bufb[p][r, pl.ds(u * 16, 16)])
                    return cr

                lax.fori_loop(0, C, addrow, 0)
                pltpu.async_copy(bufa[p], out_hbm.at[pl.ds(kof(j) * C, C)],
                                 semo[p])
            return carry

        lax.fori_loop(0, JH // 2, outer, 0)
        wait_out(JH - 2, 0)
        wait_out(JH - 1, 1)

    return gather_k


_sc_gather0 = _make_gather(0)
_sc_gather1 = _make_gather(1)


def _ln(x, g, b):
    m = jnp.mean(x, axis=-1, keepdims=True)
    xc = x - m
    v = jnp.mean(xc * xc, axis=-1, keepdims=True)
    return xc * lax.rsqrt(v + 1e-5) * g + b


def _silu(x):
    return x * jax.nn.sigmoid(x)


def _unpack_bf16_pairs(xp, nrows):
    u = jax.lax.bitcast_convert_type(xp, jnp.uint32)
    lo = jax.lax.bitcast_convert_type(u << 16, jnp.float32)
    hi = jax.lax.bitcast_convert_type(u & jnp.uint32(0xFFFF0000), jnp.float32)
    return jnp.concatenate([lo.reshape(nrows, DP), hi.reshape(nrows, DP)],
                           axis=1)


def _node_body(p_ref, n_ref, w1a, w1b, b1, g1, bb1, w2, b2, g2, bb2, ws, wd,
               nn_ref, a_ref, b_ref):
    msg = p_ref[0:NN, :] + p_ref[NN:2 * NN, :]
    nodes = n_ref[...]
    x = (jnp.dot(nodes, w1a[...], preferred_element_type=jnp.float32)
         + jnp.dot(msg, w1b[...], preferred_element_type=jnp.float32)
         + b1[...])
    x = _silu(_ln(x, g1[...], bb1[...]))
    x = jnp.dot(x, w2[...], preferred_element_type=jnp.float32) + b2[...]
    x = _silu(_ln(x, g2[...], bb2[...]))
    nn = x + nodes
    nn_ref[...] = nn
    a_ref[...] = jnp.dot(nn, ws[...], preferred_element_type=jnp.float32)
    b_ref[...] = jnp.dot(nn, wd[...], preferred_element_type=jnp.float32)


BLK = 2000


def _edge_body(g_ref, e_ref, we, b1, g1, bb1, w2, b2, g2, bb2, out_ref):
    e = e_ref[...]
    h = (g_ref[...]
         + jnp.dot(e, we[...], preferred_element_type=jnp.float32)
         + b1[...])
    h = _silu(_ln(h, g1[...], bb1[...]))
    h = jnp.dot(h, w2[...], preferred_element_type=jnp.float32) + b2[...]
    h = _ln(h, g2[...], bb2[...])
    out_ref[...] = jnp.tanh(h) + e


def _edge_body2(car_ref, g_ref, e_ref, we, b1, g1, bb1, w2, b2, g2, bb2,
                out_ref):
    del car_ref
    _edge_body(g_ref, e_ref, we, b1, g1, bb1, w2, b2, g2, bb2, out_ref)


def _row2d(v):
    return v.reshape(1, D)


def kernel(nodes, edges, node_params, edge_params, graph):
    graph = graph.astype(jnp.int32)
    src3d = graph[0].reshape(NCH, 1, C)
    dst3d = graph[1].reshape(NCH, 1, C)
    zeros = jnp.zeros((NN, D), jnp.float32)

    partials = _sc_scatter(edges, dst3d, zeros)

    np0, np1 = node_params
    ep0, ep1 = edge_params
    w1a, w1b = np0['W'][:D], np0['W'][D:]
    ws, wd, we = ep0['W'][:D], ep0['W'][D:2 * D], ep0['W'][2 * D:]

    full = pl.BlockSpec((D, D), lambda i: (0, 0))
    row = pl.BlockSpec((1, D), lambda i: (0, 0))

    nodes_new, a_arr, b_arr = pl.pallas_call(
        _node_body,
        out_shape=[jax.ShapeDtypeStruct((NN, D), jnp.float32)] * 3,
    )(partials, nodes, w1a, w1b, _row2d(np0['b']), _row2d(np0['g']),
      _row2d(np0['beta']), np1['W'], _row2d(np1['b']), _row2d(np1['g']),
      _row2d(np1['beta']), ws, wd)

    g0 = _sc_gather0(a_arr, b_arr, src3d, dst3d)
    g1 = _sc_gather1(a_arr, b_arr, src3d, dst3d)

    nblk_h = (NE // 2) // BLK
    blk = pl.BlockSpec((BLK, D), lambda i: (i, 0))
    blk_hi = pl.BlockSpec((BLK, D), lambda i: (i + nblk_h, 0))
    ewts = (we, _row2d(ep0['b']), _row2d(ep0['g']), _row2d(ep0['beta']),
            ep1['W'], _row2d(ep1['b']), _row2d(ep1['g']), _row2d(ep1['beta']))
    wspecs = [full, row, row, row, full, row, row, row]

    o0 = pl.pallas_call(
        _edge_body,
        grid=(nblk_h,),
        in_specs=[blk, blk] + wspecs,
        out_specs=blk,
        out_shape=jax.ShapeDtypeStruct((NE, D), jnp.float32),
    )(g0, edges, *ewts)

    edges_new = pl.pallas_call(
        _edge_body2,
        grid=(nblk_h,),
        in_specs=[pl.BlockSpec(memory_space=pl.ANY), blk, blk_hi] + wspecs,
        out_specs=blk_hi,
        out_shape=jax.ShapeDtypeStruct((NE, D), jnp.float32),
        input_output_aliases={0: 0},
    )(o0, g1, edges, *ewts)

    return nodes_new, edges_new
